# baseline (speedup 1.0000x reference)
import functools

import jax
import jax.numpy as jnp
from jax import lax
from jax.experimental import pallas as pl
from jax.experimental.pallas import tpu as pltpu
from jax.experimental.pallas import tpu_sc as plsc

V = 1000
NB = 1024
T = 50
NC = 2
NS = 16
NW = NC * NS
DBLK = 40
NDB = V // DBLK
UNITS = NDB * T
NG = NB // 16

_mesh = plsc.VectorSubcoreMesh(core_axis_name="c", subcore_axis_name="s")


@functools.partial(
    pl.kernel,
    mesh=_mesh,
    out_type=jax.ShapeDtypeStruct((T, V, NB), jnp.float32),
    scratch_types=[
        pltpu.VMEM((NB,), jnp.int32),
        pltpu.VMEM((DBLK, V), jnp.float32),
        pltpu.VMEM((DBLK, NB), jnp.float32),
        pltpu.VMEM((DBLK, NB), jnp.float32),
        pltpu.SemaphoreType.DMA,
        pltpu.SemaphoreType.DMA,
    ],
    compiler_params=pltpu.CompilerParams(needs_layout_passes=False),
)
def _gather_t(idxT_hbm, tableT_hbm, out_hbm, idx_v, slab_v, buf_a, buf_b,
              wsem_a, wsem_b):
    wid = lax.axis_index("s") * NC + lax.axis_index("c")
    u0 = wid * UNITS // NW
    u1 = (wid + 1) * UNITS // NW
    cnt = u1 - u0
    npairs = (cnt + 1) // 2

    def do_unit(i, buf, sem):
        u = u0 + i
        dblk = u // T
        t = u - dblk * T
        d0 = dblk * DBLK

        @pl.when((t == 0) | (i == 0))
        def _():
            pltpu.sync_copy(tableT_hbm.at[pl.ds(d0, DBLK)], slab_v)

        pltpu.sync_copy(idxT_hbm.at[t], idx_v)

        @pl.when(i >= 2)
        def _():
            pltpu.make_async_copy(buf, out_hbm.at[t, pl.ds(d0, DBLK)],
                                  sem).wait()

        def g_body(g, carry):
            col16 = idx_v[pl.ds(16 * g, 16)]
            for d in range(DBLK):
                x = plsc.load_gather(
                    slab_v, [jnp.full((16,), d, jnp.int32), col16])
                buf[d, pl.ds(16 * g, 16)] = x
            return carry

        lax.fori_loop(0, NG, g_body, 0)
        pltpu.async_copy(buf, out_hbm.at[t, pl.ds(d0, DBLK)], sem)

    def pair_body(q, carry):
        i = 2 * q

        @pl.when(i < cnt)
        def _():
            do_unit(i, buf_a, wsem_a)

        @pl.when(i + 1 < cnt)
        def _():
            do_unit(i + 1, buf_b, wsem_b)

        return carry

    lax.fori_loop(0, npairs, pair_body, 0)

    pltpu.make_async_copy(buf_a, out_hbm.at[0, pl.ds(0, DBLK)], wsem_a).wait()
    pltpu.make_async_copy(buf_b, out_hbm.at[0, pl.ds(0, DBLK)], wsem_b).wait()


def kernel(idx, table):
    idxT = idx.astype(jnp.int32).T
    tableT = table.T
    out_t = _gather_t(idxT, tableT)
    return jnp.transpose(out_t, (2, 0, 1))


# speedup vs baseline: 1.2324x; 1.2324x over previous
import functools

import jax
import jax.numpy as jnp
from jax import lax
from jax.experimental import pallas as pl
from jax.experimental.pallas import tpu as pltpu
from jax.experimental.pallas import tpu_sc as plsc

D = 1000
DM = 896
DT = 128
TAIL = D - DM
NC = 2
NS = 16
NW = NC * NS
NB = 1024
T = 50
TA = 48
TR = T - TA
B_PER_W = NB // NW
N_PAIRS = B_PER_W // 2

_mesh = plsc.VectorSubcoreMesh(core_axis_name="c", subcore_axis_name="s")


@functools.partial(
    pl.kernel,
    mesh=_mesh,
    out_type=jax.ShapeDtypeStruct((NB, T, D), jnp.float32),
    scratch_types=[
        pltpu.VMEM((B_PER_W, T), jnp.int32),
        pltpu.VMEM((T, D), jnp.float32),
        pltpu.VMEM((T, D), jnp.float32),
        pltpu.VMEM((TR, DM), jnp.float32),
        pltpu.VMEM((T, DT), jnp.float32),
        pltpu.SemaphoreType.DMA,
        pltpu.SemaphoreType.DMA,
        pltpu.SemaphoreType.DMA,
        pltpu.SemaphoreType.DMA,
        pltpu.SemaphoreType.DMA,
        pltpu.SemaphoreType.DMA,
    ],
    compiler_params=pltpu.CompilerParams(needs_layout_passes=False),
)
def _gather_rows(idx_hbm, head_hbm, tail_hbm, out_hbm, idx_v, buf_a, buf_b,
                 buf_r, buf_t, gsem_a, gsem_b, wsem_a, wsem_b, rsem, tsem):
    wid = lax.axis_index("s") * NC + lax.axis_index("c")
    base = wid * B_PER_W
    pltpu.sync_copy(idx_hbm.at[pl.ds(base, B_PER_W)], idx_v)

    def g_start(j, buf, sem):
        pltpu.async_copy(
            head_hbm.at[idx_v.at[j, pl.ds(0, TA)]],
            buf.at[pl.ds(0, TA), pl.ds(0, DM)], sem)

    def g_wait(j, buf, sem):
        pltpu.make_async_copy(
            head_hbm.at[idx_v.at[j, pl.ds(0, TA)]],
            buf.at[pl.ds(0, TA), pl.ds(0, DM)], sem).wait()

    def r_start(j):
        pltpu.async_copy(head_hbm.at[idx_v.at[j, pl.ds(TA, TR)]], buf_r, rsem)

    def r_wait(j):
        pltpu.make_async_copy(
            head_hbm.at[idx_v.at[j, pl.ds(TA, TR)]], buf_r, rsem).wait()

    def t_start(j):
        pltpu.async_copy(tail_hbm.at[idx_v.at[j]], buf_t, tsem)

    def t_wait(j):
        pltpu.make_async_copy(tail_hbm.at[idx_v.at[j]], buf_t, tsem).wait()

    def w_start(j, buf, sem):
        pltpu.async_copy(buf, out_hbm.at[base + j], sem)

    def w_wait(j, buf, sem):
        pltpu.make_async_copy(buf, out_hbm.at[base + j], sem).wait()

    col8 = DM + 96 + lax.iota(jnp.int32, 16)
    msk8 = lax.iota(jnp.int32, 16) < (TAIL - 96)

    def fix_head(buf):
        for i in range(TR):
            for k in range(DM // 16):
                buf[TA + i, pl.ds(16 * k, 16)] = buf_r[i, pl.ds(16 * k, 16)]

    def fix_tail(buf):
        for r in range(T):
            for k in range(6):
                buf[r, pl.ds(DM + 16 * k, 16)] = buf_t[r, pl.ds(16 * k, 16)]
            x = buf_t[r, pl.ds(96, 16)]
            plsc.store_scatter(buf, [jnp.full((16,), r, jnp.int32), col8], x,
                               mask=msk8)

    g_start(0, buf_a, gsem_a)
    r_start(0)
    t_start(0)
    g_start(1, buf_b, gsem_b)

    def half(j, buf, gsem, wsem, prefetch):
        g_wait(j, buf, gsem)
        r_wait(j)
        fix_head(buf)
        t_wait(j)
        fix_tail(buf)
        if prefetch:
            r_start(j + 1)
            t_start(j + 1)
        w_start(j, buf, wsem)

    def pair_body(p, carry):
        j = 2 * p
        half(j, buf_a, gsem_a, wsem_a, True)
        half(j + 1, buf_b, gsem_b, wsem_b, True)
        w_wait(j, buf_a, wsem_a)
        g_start(j + 2, buf_a, gsem_a)
        w_wait(j + 1, buf_b, wsem_b)
        g_start(j + 3, buf_b, gsem_b)
        return carry

    lax.fori_loop(0, N_PAIRS - 1, pair_body, 0)

    j = B_PER_W - 2
    half(j, buf_a, gsem_a, wsem_a, True)
    half(j + 1, buf_b, gsem_b, wsem_b, False)
    w_wait(j, buf_a, wsem_a)
    w_wait(j + 1, buf_b, wsem_b)


def kernel(idx, table):
    head = table[:, :DM]
    tail = jnp.pad(table[:, DM:], ((0, 0), (0, DT - TAIL)))
    return _gather_rows(idx.astype(jnp.int32), head, tail)
